# Initial kernel scaffold; baseline (speedup 1.0000x reference)
import functools

import jax
import jax.numpy as jnp
from jax.experimental import pallas as pl
from jax.experimental.pallas import tpu as pltpu

H = 1024
F = 2048
E = 8
N = 2048
EPAD = 128


def _router_body(x_ref, rw_ref, comb_ref):
    logits = jnp.dot(x_ref[...], rw_ref[...], preferred_element_type=jnp.float32)
    col = jax.lax.broadcasted_iota(jnp.int32, (N, EPAD), 1)
    valid = col < E
    masked = jnp.where(valid, logits, -1e30)
    m1 = jnp.max(masked, axis=1, keepdims=True)
    i1 = jnp.min(jnp.where(masked == m1, col, EPAD), axis=1, keepdims=True)
    masked2 = jnp.where(col == i1, -1e30, masked)
    m2 = jnp.max(masked2, axis=1, keepdims=True)
    i2 = jnp.min(jnp.where(masked2 == m2, col, EPAD), axis=1, keepdims=True)
    g1 = 1.0 / (1.0 + jnp.exp(m2 - m1))
    g2 = 1.0 - g1
    comb_ref[...] = jnp.where(col == i1, g1, 0.0) + jnp.where(col == i2, g2, 0.0)


def _expert_body(x_ref, w1_ref, w2_ref, comb_ref, out_ref):
    e = pl.program_id(0)
    h = jax.nn.gelu(jnp.dot(x_ref[...], w1_ref[...],
                            preferred_element_type=jnp.float32))
    y = jnp.dot(h, w2_ref[...], preferred_element_type=jnp.float32)
    col = jax.lax.broadcasted_iota(jnp.int32, (N, EPAD), 1)
    c_e = jnp.sum(jnp.where(col == e, comb_ref[...], 0.0), axis=1, keepdims=True)

    @pl.when(e == 0)
    def _init():
        out_ref[...] = c_e * y

    @pl.when(e != 0)
    def _acc():
        out_ref[...] = out_ref[...] + c_e * y


def kernel(x, router_weight, w1, w2):
    rw_pad = jnp.pad(router_weight, ((0, 0), (0, EPAD - E)))
    combine = pl.pallas_call(
        _router_body,
        out_shape=jax.ShapeDtypeStruct((N, EPAD), jnp.float32),
    )(x, rw_pad)

    out = pl.pallas_call(
        _expert_body,
        grid=(E,),
        in_specs=[
            pl.BlockSpec((N, H), lambda e: (0, 0)),
            pl.BlockSpec((H, F), lambda e: (0, e)),
            pl.BlockSpec((F, H), lambda e: (e, 0)),
            pl.BlockSpec((N, EPAD), lambda e: (0, 0)),
        ],
        out_specs=pl.BlockSpec((N, H), lambda e: (0, 0)),
        out_shape=jax.ShapeDtypeStruct((N, H), jnp.float32),
    )(x, w1, w2, combine)
    return out


# speedup vs baseline: 2.3816x; 2.3816x over previous
import functools

import jax
import jax.numpy as jnp
from jax.experimental import pallas as pl
from jax.experimental.pallas import tpu as pltpu

H = 1024
F = 2048
E = 8
N = 2048
EPAD = 128


def _router_body(x_ref, rw_ref, comb_ref):
    logits = jnp.dot(x_ref[...], rw_ref[...], preferred_element_type=jnp.float32)
    col = jax.lax.broadcasted_iota(jnp.int32, (N, EPAD), 1)
    valid = col < E
    masked = jnp.where(valid, logits, -1e30)
    m1 = jnp.max(masked, axis=1, keepdims=True)
    i1 = jnp.min(jnp.where(masked == m1, col, EPAD), axis=1, keepdims=True)
    masked2 = jnp.where(col == i1, -1e30, masked)
    m2 = jnp.max(masked2, axis=1, keepdims=True)
    i2 = jnp.min(jnp.where(masked2 == m2, col, EPAD), axis=1, keepdims=True)
    g1 = 1.0 / (1.0 + jnp.exp(m2 - m1))
    g2 = 1.0 - g1
    comb_ref[...] = jnp.where(col == i1, g1, 0.0) + jnp.where(col == i2, g2, 0.0)


FC = 512


def _expert_body(x_ref, w1_ref, w2_ref, comb_ref, out_ref):
    e = pl.program_id(0)
    f = pl.program_id(1)
    h = jax.nn.gelu(jnp.dot(x_ref[...], w1_ref[...],
                            preferred_element_type=jnp.float32))
    y = jnp.dot(h, w2_ref[...], preferred_element_type=jnp.float32)
    col = jax.lax.broadcasted_iota(jnp.int32, (N, EPAD), 1)
    c_e = jnp.sum(jnp.where(col == e, comb_ref[...], 0.0), axis=1, keepdims=True)

    @pl.when((e == 0) & (f == 0))
    def _init():
        out_ref[...] = c_e * y

    @pl.when((e != 0) | (f != 0))
    def _acc():
        out_ref[...] = out_ref[...] + c_e * y


def kernel(x, router_weight, w1, w2):
    rw_pad = jnp.pad(router_weight, ((0, 0), (0, EPAD - E)))
    combine = pl.pallas_call(
        _router_body,
        out_shape=jax.ShapeDtypeStruct((N, EPAD), jnp.float32),
    )(x, rw_pad)

    nfc = F // FC
    out = pl.pallas_call(
        _expert_body,
        grid=(E, nfc),
        in_specs=[
            pl.BlockSpec((N, H), lambda e, f: (0, 0)),
            pl.BlockSpec((H, FC), lambda e, f: (0, e * nfc + f)),
            pl.BlockSpec((FC, H), lambda e, f: (e * nfc + f, 0)),
            pl.BlockSpec((N, EPAD), lambda e, f: (0, 0)),
        ],
        out_specs=pl.BlockSpec((N, H), lambda e, f: (0, 0)),
        out_shape=jax.ShapeDtypeStruct((N, H), jnp.float32),
    )(x, w1, w2, combine)
    return out


# speedup vs baseline: 2.8837x; 1.2108x over previous
import jax
import jax.numpy as jnp
from jax import lax
from jax.experimental import pallas as pl
from jax.experimental.pallas import tpu as pltpu
from jax.experimental.pallas import tpu_sc as plsc

H = 1024
F = 2048
E = 8
N = 2048
EPAD = 128

B = 256
P = 2 * N + E * B
NB = P // B

RT = 32
TPT = N // RT
L = 16


def _splat(v, dtype=jnp.int32):
    return jnp.full((L,), v, dtype)


def _router_body(x_ref, rw_ref, logits_ref, start_ref, blk_ref):
    logits = jnp.dot(x_ref[...], rw_ref[...], preferred_element_type=jnp.float32)
    logits_ref[...] = logits
    col = lax.broadcasted_iota(jnp.int32, (N, EPAD), 1)
    masked = jnp.where(col < E, logits, -1e30)
    m1 = jnp.max(masked, axis=1, keepdims=True)
    i1 = jnp.min(jnp.where(masked == m1, col, EPAD), axis=1, keepdims=True)
    masked2 = jnp.where(col == i1, -1e30, masked)
    m2 = jnp.max(masked2, axis=1, keepdims=True)
    i2 = jnp.min(jnp.where(masked2 == m2, col, EPAD), axis=1, keepdims=True)
    onehot = ((col == i1).astype(jnp.float32) +
              (col == i2).astype(jnp.float32))

    acc = jnp.zeros((1, EPAD), jnp.float32)
    pre_rows = []
    for w in range(RT):
        pre_rows.append(acc)
        crow = jnp.sum(onehot[w * TPT:(w + 1) * TPT, :], axis=0, keepdims=True)
        acc = acc + crow
    total = acc
    padded = jnp.ceil(total * (1.0 / B)) * B

    colr = lax.broadcasted_iota(jnp.int32, (1, EPAD), 1)
    excl = jnp.zeros((1, EPAD), jnp.float32)
    running = jnp.float32(0.0)
    incls = []
    for e in range(E):
        pe = jnp.sum(jnp.where(colr == e, padded, 0.0))
        excl = excl + jnp.where(colr == e, running, 0.0)
        running = running + pe
        incls.append(running)

    start = excl + jnp.concatenate(pre_rows, axis=0)
    start_ref[...] = start.astype(jnp.int32)

    bixf = colr.astype(jnp.float32) * B
    be_row = jnp.zeros((1, EPAD), jnp.float32)
    for e in range(E):
        be_row = be_row + jnp.where(incls[e] <= bixf, 1.0, 0.0)
    be_row = jnp.minimum(be_row, E - 1)
    bu_row = jnp.where(bixf < running, 1.0, 0.0)
    zrows = jnp.zeros((6, EPAD), jnp.float32)
    blk_ref[...] = jnp.concatenate([be_row, bu_row, zrows],
                                   axis=0).astype(jnp.int32)


def _router(x, rw_pad):
    return pl.pallas_call(
        _router_body,
        out_shape=[
            jax.ShapeDtypeStruct((N, EPAD), jnp.float32),
            jax.ShapeDtypeStruct((RT, EPAD), jnp.int32),
            jax.ShapeDtypeStruct((8, EPAD), jnp.int32),
        ],
    )(x, rw_pad)


def _route_body(logits_hbm, start_hbm, x_hbm, z_hbm,
                xs_hbm, pos0_hbm, pos1_hbm, g0_hbm, g1_hbm,
                logit_v, x_v, idx0_v, idx1_v, g0_v, g1_v, srow_v, z_v, sem):
    wid = lax.axis_index("s") * 2 + lax.axis_index("c")
    base = wid * TPT

    pltpu.sync_copy(logits_hbm.at[pl.ds(base * EPAD, TPT * EPAD)], logit_v)
    pltpu.sync_copy(start_hbm.at[wid], srow_v)
    pltpu.sync_copy(z_hbm, z_v)
    zvec = z_v[...]

    iota = lax.iota(jnp.int32, L)
    e0s, e1s = [], []
    for g in range(TPT // L):
        rowoff = (iota + _splat(g * L)) * EPAD
        vals = [plsc.load_gather(logit_v, [rowoff + _splat(e)])
                for e in range(E)]
        m1 = vals[0]
        for v in vals[1:]:
            m1 = jnp.maximum(m1, v)
        i1 = _splat(E)
        for e in range(E):
            i1 = jnp.minimum(i1,
                             jnp.where(vals[e] == m1, _splat(e), _splat(E)))
        m2 = _splat(-1e30, jnp.float32)
        for e in range(E):
            m2 = jnp.maximum(m2, jnp.where(i1 == _splat(e),
                                           _splat(-1e30, jnp.float32), vals[e]))
        i2 = _splat(E)
        for e in range(E):
            hit = (vals[e] == m2) & (i1 != _splat(e))
            i2 = jnp.minimum(i2, jnp.where(hit, _splat(e), _splat(E)))
        ga = 1.0 / (1.0 + jnp.exp(m2 - m1))
        e0s.append(i1)
        e1s.append(i2)
        g0_v[pl.ds(g * L, L)] = ga
        g1_v[pl.ds(g * L, L)] = 1.0 - ga

    run = [plsc.load_gather(srow_v, [zvec + _splat(e)]) for e in range(E)]
    for vlist, dst in ((e0s, idx0_v), (e1s, idx1_v)):
        for g, v in enumerate(vlist):
            pos = _splat(0)
            for e in range(E):
                mask = v == _splat(e)
                prefix = plsc.cumsum(mask.astype(jnp.int32))
                pos = pos + jnp.where(mask, run[e] + prefix - _splat(1),
                                      _splat(0))
                run[e] = run[e] + plsc.all_reduce_population_count(mask)
            dst[pl.ds(g * L, L)] = pos

    pltpu.sync_copy(idx0_v, pos0_hbm.at[pl.ds(base, TPT)])
    pltpu.sync_copy(idx1_v, pos1_hbm.at[pl.ds(base, TPT)])
    pltpu.sync_copy(g0_v, g0_hbm.at[pl.ds(base, TPT)])
    pltpu.sync_copy(g1_v, g1_hbm.at[pl.ds(base, TPT)])

    pltpu.sync_copy(x_hbm.at[pl.ds(base, TPT)], x_v)
    d0 = pltpu.async_copy(x_v, xs_hbm.at[idx0_v], sem)
    d0.wait()
    d1 = pltpu.async_copy(x_v, xs_hbm.at[idx1_v], sem)
    d1.wait()


def _route_scatter(logits, start, x, zeros16):
    mesh = plsc.VectorSubcoreMesh(core_axis_name="c", subcore_axis_name="s")
    return pl.kernel(
        _route_body,
        out_type=[
            jax.ShapeDtypeStruct((P, H), jnp.float32),
            jax.ShapeDtypeStruct((N,), jnp.int32),
            jax.ShapeDtypeStruct((N,), jnp.int32),
            jax.ShapeDtypeStruct((N,), jnp.float32),
            jax.ShapeDtypeStruct((N,), jnp.float32),
        ],
        mesh=mesh,
        compiler_params=pltpu.CompilerParams(needs_layout_passes=False),
        scratch_types=[
            pltpu.VMEM((TPT * EPAD,), jnp.float32),
            pltpu.VMEM((TPT, H), jnp.float32),
            pltpu.VMEM((TPT,), jnp.int32),
            pltpu.VMEM((TPT,), jnp.int32),
            pltpu.VMEM((TPT,), jnp.float32),
            pltpu.VMEM((TPT,), jnp.float32),
            pltpu.VMEM((EPAD,), jnp.int32),
            pltpu.VMEM((L,), jnp.int32),
            pltpu.SemaphoreType.DMA,
        ],
    )(logits, start, x, zeros16)


def _group_body(be_ref, bu_ref, x_ref, w1_ref, w2_ref, y_ref):
    b = pl.program_id(0)

    @pl.when(bu_ref[b] == 1)
    def _():
        h = jax.nn.gelu(jnp.dot(x_ref[...], w1_ref[...],
                                preferred_element_type=jnp.float32))
        y_ref[...] = jnp.dot(h, w2_ref[...], preferred_element_type=jnp.float32)


def _grouped_ffn(be, bu, x_sorted, w1, w2):
    grid_spec = pltpu.PrefetchScalarGridSpec(
        num_scalar_prefetch=2,
        grid=(NB,),
        in_specs=[
            pl.BlockSpec((B, H), lambda b, be, bu: (b, 0)),
            pl.BlockSpec((H, F), lambda b, be, bu: (0, be[b])),
            pl.BlockSpec((F, H), lambda b, be, bu: (be[b], 0)),
        ],
        out_specs=pl.BlockSpec((B, H), lambda b, be, bu: (b, 0)),
    )
    return pl.pallas_call(
        _group_body,
        grid_spec=grid_spec,
        out_shape=jax.ShapeDtypeStruct((P, H), jnp.float32),
    )(be, bu, x_sorted, w1, w2)


CCH = 16


def _combine_body(y_hbm, pos0_hbm, pos1_hbm, g0_hbm, g1_hbm, z_hbm, out_hbm,
                  idx0_v, idx1_v, g0_v, g1_v, buf0, buf1, outb, z_v, sem):
    wid = lax.axis_index("s") * 2 + lax.axis_index("c")
    base = wid * TPT
    pltpu.sync_copy(pos0_hbm.at[pl.ds(base, TPT)], idx0_v)
    pltpu.sync_copy(pos1_hbm.at[pl.ds(base, TPT)], idx1_v)
    pltpu.sync_copy(g0_hbm.at[pl.ds(base, TPT)], g0_v)
    pltpu.sync_copy(g1_hbm.at[pl.ds(base, TPT)], g1_v)
    pltpu.sync_copy(z_hbm, z_v)
    zvec = z_v[...]

    for c in range(TPT // CCH):
        d0 = pltpu.async_copy(y_hbm.at[idx0_v.at[pl.ds(c * CCH, CCH)]], buf0,
                              sem)
        d1 = pltpu.async_copy(y_hbm.at[idx1_v.at[pl.ds(c * CCH, CCH)]], buf1,
                              sem)
        d0.wait()
        d1.wait()
        for i in range(CCH):
            t = zvec + _splat(c * CCH + i)
            g0s = plsc.load_gather(g0_v, [t])
            g1s = plsc.load_gather(g1_v, [t])

            def body(j, carry, i=i, g0s=g0s, g1s=g1s):
                sl = pl.ds(j * L, L)
                outb[i, sl] = g0s * buf0[i, sl] + g1s * buf1[i, sl]
                return carry

            lax.fori_loop(0, H // L, body, 0)
        pltpu.sync_copy(outb, out_hbm.at[pl.ds(base + c * CCH, CCH)])


def _combine(y_sorted, pos0, pos1, g0, g1, zeros16):
    mesh = plsc.VectorSubcoreMesh(core_axis_name="c", subcore_axis_name="s")
    return pl.kernel(
        _combine_body,
        out_type=jax.ShapeDtypeStruct((N, H), jnp.float32),
        mesh=mesh,
        compiler_params=pltpu.CompilerParams(needs_layout_passes=False),
        scratch_types=[
            pltpu.VMEM((TPT,), jnp.int32),
            pltpu.VMEM((TPT,), jnp.int32),
            pltpu.VMEM((TPT,), jnp.float32),
            pltpu.VMEM((TPT,), jnp.float32),
            pltpu.VMEM((CCH, H), jnp.float32),
            pltpu.VMEM((CCH, H), jnp.float32),
            pltpu.VMEM((CCH, H), jnp.float32),
            pltpu.VMEM((L,), jnp.int32),
            pltpu.SemaphoreType.DMA,
        ],
    )(y_sorted, pos0, pos1, g0, g1, zeros16)


def kernel(x, router_weight, w1, w2):
    rw_pad = jnp.pad(router_weight, ((0, 0), (0, EPAD - E)))
    zeros16 = jnp.zeros((L,), jnp.int32)
    logits, start, blk = _router(x, rw_pad)
    be = blk[0, :NB]
    bu = blk[1, :NB]
    x_sorted, pos0, pos1, g0, g1 = _route_scatter(
        logits.reshape(N * EPAD), start, x, zeros16)
    y_sorted = _grouped_ffn(be, bu, x_sorted, w1, w2)
    return _combine(y_sorted, pos0, pos1, g0, g1, zeros16)


# speedup vs baseline: 2.9033x; 1.0068x over previous
import jax
import jax.numpy as jnp
from jax import lax
from jax.experimental import pallas as pl
from jax.experimental.pallas import tpu as pltpu
from jax.experimental.pallas import tpu_sc as plsc

H = 1024
F = 2048
E = 8
N = 2048
EPAD = 128

B = 256
P = 2 * N + E * B
NB = P // B

RT = 32
TPT = N // RT
L = 16


def _splat(v, dtype=jnp.int32):
    return jnp.full((L,), v, dtype)


def _router_body(x_ref, rw_ref, logits_ref, start_ref, blk_ref):
    logits = jnp.dot(x_ref[...], rw_ref[...], preferred_element_type=jnp.float32)
    logits_ref[...] = logits
    col = lax.broadcasted_iota(jnp.int32, (N, EPAD), 1)
    masked = jnp.where(col < E, logits, -1e30)
    m1 = jnp.max(masked, axis=1, keepdims=True)
    i1 = jnp.min(jnp.where(masked == m1, col, EPAD), axis=1, keepdims=True)
    masked2 = jnp.where(col == i1, -1e30, masked)
    m2 = jnp.max(masked2, axis=1, keepdims=True)
    i2 = jnp.min(jnp.where(masked2 == m2, col, EPAD), axis=1, keepdims=True)
    onehot = ((col == i1).astype(jnp.float32) +
              (col == i2).astype(jnp.float32))

    acc = jnp.zeros((1, EPAD), jnp.float32)
    pre_rows = []
    for w in range(RT):
        pre_rows.append(acc)
        crow = jnp.sum(onehot[w * TPT:(w + 1) * TPT, :], axis=0, keepdims=True)
        acc = acc + crow
    total = acc
    padded = jnp.ceil(total * (1.0 / B)) * B

    colr = lax.broadcasted_iota(jnp.int32, (1, EPAD), 1)
    excl = jnp.zeros((1, EPAD), jnp.float32)
    running = jnp.float32(0.0)
    incls = []
    for e in range(E):
        pe = jnp.sum(jnp.where(colr == e, padded, 0.0))
        excl = excl + jnp.where(colr == e, running, 0.0)
        running = running + pe
        incls.append(running)

    start = excl + jnp.concatenate(pre_rows, axis=0)
    start_ref[...] = start.astype(jnp.int32)

    bixf = colr.astype(jnp.float32) * B
    be_row = jnp.zeros((1, EPAD), jnp.float32)
    for e in range(E):
        be_row = be_row + jnp.where(incls[e] <= bixf, 1.0, 0.0)
    be_row = jnp.minimum(be_row, E - 1)
    bu_row = jnp.where(bixf < running, 1.0, 0.0)
    zrows = jnp.zeros((6, EPAD), jnp.float32)
    blk_ref[...] = jnp.concatenate([be_row, bu_row, zrows],
                                   axis=0).astype(jnp.int32)


def _router(x, rw_pad):
    return pl.pallas_call(
        _router_body,
        out_shape=[
            jax.ShapeDtypeStruct((N, EPAD), jnp.float32),
            jax.ShapeDtypeStruct((RT, EPAD), jnp.int32),
            jax.ShapeDtypeStruct((8, EPAD), jnp.int32),
        ],
    )(x, rw_pad)


def _route_body(logits_hbm, start_hbm, x_hbm, z_hbm,
                xs_hbm, pos0_hbm, pos1_hbm, g0_hbm, g1_hbm,
                logit_v, x_v, idx0_v, idx1_v, g0_v, g1_v, srow_v, z_v,
                sem, xsem):
    wid = lax.axis_index("s") * 2 + lax.axis_index("c")
    base = wid * TPT

    pltpu.sync_copy(logits_hbm.at[pl.ds(base * EPAD, TPT * EPAD)], logit_v)
    xload = pltpu.async_copy(x_hbm.at[pl.ds(base, TPT)], x_v, xsem)
    pltpu.sync_copy(start_hbm.at[wid], srow_v)
    pltpu.sync_copy(z_hbm, z_v)
    zvec = z_v[...]

    iota = lax.iota(jnp.int32, L)
    e0s, e1s = [], []
    for g in range(TPT // L):
        rowoff = (iota + _splat(g * L)) * EPAD
        vals = [plsc.load_gather(logit_v, [rowoff + _splat(e)])
                for e in range(E)]
        m1 = vals[0]
        for v in vals[1:]:
            m1 = jnp.maximum(m1, v)
        i1 = _splat(E)
        for e in range(E):
            i1 = jnp.minimum(i1,
                             jnp.where(vals[e] == m1, _splat(e), _splat(E)))
        m2 = _splat(-1e30, jnp.float32)
        for e in range(E):
            m2 = jnp.maximum(m2, jnp.where(i1 == _splat(e),
                                           _splat(-1e30, jnp.float32), vals[e]))
        i2 = _splat(E)
        for e in range(E):
            hit = (vals[e] == m2) & (i1 != _splat(e))
            i2 = jnp.minimum(i2, jnp.where(hit, _splat(e), _splat(E)))
        ga = 1.0 / (1.0 + jnp.exp(m2 - m1))
        e0s.append(i1)
        e1s.append(i2)
        g0_v[pl.ds(g * L, L)] = ga
        g1_v[pl.ds(g * L, L)] = 1.0 - ga

    run = [plsc.load_gather(srow_v, [zvec + _splat(e)]) for e in range(E)]
    for vlist, dst in ((e0s, idx0_v), (e1s, idx1_v)):
        for g, v in enumerate(vlist):
            pos = _splat(0)
            for e in range(E):
                mask = v == _splat(e)
                prefix = plsc.cumsum(mask.astype(jnp.int32))
                pos = pos + jnp.where(mask, run[e] + prefix - _splat(1),
                                      _splat(0))
                run[e] = run[e] + plsc.all_reduce_population_count(mask)
            dst[pl.ds(g * L, L)] = pos

    pltpu.sync_copy(idx0_v, pos0_hbm.at[pl.ds(base, TPT)])
    pltpu.sync_copy(idx1_v, pos1_hbm.at[pl.ds(base, TPT)])
    pltpu.sync_copy(g0_v, g0_hbm.at[pl.ds(base, TPT)])
    pltpu.sync_copy(g1_v, g1_hbm.at[pl.ds(base, TPT)])

    xload.wait()
    d0 = pltpu.async_copy(x_v, xs_hbm.at[idx0_v], sem)
    d1 = pltpu.async_copy(x_v, xs_hbm.at[idx1_v], sem)
    d0.wait()
    d1.wait()


def _route_scatter(logits, start, x, zeros16):
    mesh = plsc.VectorSubcoreMesh(core_axis_name="c", subcore_axis_name="s")
    return pl.kernel(
        _route_body,
        out_type=[
            jax.ShapeDtypeStruct((P, H), jnp.float32),
            jax.ShapeDtypeStruct((N,), jnp.int32),
            jax.ShapeDtypeStruct((N,), jnp.int32),
            jax.ShapeDtypeStruct((N,), jnp.float32),
            jax.ShapeDtypeStruct((N,), jnp.float32),
        ],
        mesh=mesh,
        compiler_params=pltpu.CompilerParams(needs_layout_passes=False),
        scratch_types=[
            pltpu.VMEM((TPT * EPAD,), jnp.float32),
            pltpu.VMEM((TPT, H), jnp.float32),
            pltpu.VMEM((TPT,), jnp.int32),
            pltpu.VMEM((TPT,), jnp.int32),
            pltpu.VMEM((TPT,), jnp.float32),
            pltpu.VMEM((TPT,), jnp.float32),
            pltpu.VMEM((EPAD,), jnp.int32),
            pltpu.VMEM((L,), jnp.int32),
            pltpu.SemaphoreType.DMA,
            pltpu.SemaphoreType.DMA,
        ],
    )(logits, start, x, zeros16)


def _group_body(be_ref, bu_ref, x_ref, w1_ref, w2_ref, y_ref):
    b = pl.program_id(0)

    @pl.when(bu_ref[b] == 1)
    def _():
        h = jax.nn.gelu(jnp.dot(x_ref[...], w1_ref[...],
                                preferred_element_type=jnp.float32,
                                precision=lax.Precision.DEFAULT))
        y_ref[...] = jnp.dot(h, w2_ref[...], preferred_element_type=jnp.float32,
                             precision=lax.Precision.DEFAULT)


def _grouped_ffn(be, bu, x_sorted, w1, w2):
    grid_spec = pltpu.PrefetchScalarGridSpec(
        num_scalar_prefetch=2,
        grid=(NB,),
        in_specs=[
            pl.BlockSpec((B, H), lambda b, be, bu: (b, 0)),
            pl.BlockSpec((H, F), lambda b, be, bu: (0, be[b])),
            pl.BlockSpec((F, H), lambda b, be, bu: (be[b], 0)),
        ],
        out_specs=pl.BlockSpec((B, H), lambda b, be, bu: (b, 0)),
    )
    return pl.pallas_call(
        _group_body,
        grid_spec=grid_spec,
        out_shape=jax.ShapeDtypeStruct((P, H), jnp.float32),
    )(be, bu, x_sorted, w1, w2)


CCH = 16


def _combine_body(y_hbm, pos0_hbm, pos1_hbm, g0_hbm, g1_hbm, z_hbm, out_hbm,
                  idx0_v, idx1_v, g0_v, g1_v, buf0, buf1, outb, z_v, sem):
    wid = lax.axis_index("s") * 2 + lax.axis_index("c")
    base = wid * TPT
    pltpu.sync_copy(pos0_hbm.at[pl.ds(base, TPT)], idx0_v)
    pltpu.sync_copy(pos1_hbm.at[pl.ds(base, TPT)], idx1_v)
    pltpu.sync_copy(g0_hbm.at[pl.ds(base, TPT)], g0_v)
    pltpu.sync_copy(g1_hbm.at[pl.ds(base, TPT)], g1_v)
    pltpu.sync_copy(z_hbm, z_v)
    zvec = z_v[...]

    for c in range(TPT // CCH):
        d0 = pltpu.async_copy(y_hbm.at[idx0_v.at[pl.ds(c * CCH, CCH)]], buf0,
                              sem)
        d1 = pltpu.async_copy(y_hbm.at[idx1_v.at[pl.ds(c * CCH, CCH)]], buf1,
                              sem)
        d0.wait()
        d1.wait()
        for i in range(CCH):
            t = zvec + _splat(c * CCH + i)
            g0s = plsc.load_gather(g0_v, [t])
            g1s = plsc.load_gather(g1_v, [t])

            def body(j, carry, i=i, g0s=g0s, g1s=g1s):
                sl = pl.ds(j * L, L)
                outb[i, sl] = g0s * buf0[i, sl] + g1s * buf1[i, sl]
                return carry

            lax.fori_loop(0, H // L, body, 0)
        pltpu.sync_copy(outb, out_hbm.at[pl.ds(base + c * CCH, CCH)])


def _combine(y_sorted, pos0, pos1, g0, g1, zeros16):
    mesh = plsc.VectorSubcoreMesh(core_axis_name="c", subcore_axis_name="s")
    return pl.kernel(
        _combine_body,
        out_type=jax.ShapeDtypeStruct((N, H), jnp.float32),
        mesh=mesh,
        compiler_params=pltpu.CompilerParams(needs_layout_passes=False),
        scratch_types=[
            pltpu.VMEM((TPT,), jnp.int32),
            pltpu.VMEM((TPT,), jnp.int32),
            pltpu.VMEM((TPT,), jnp.float32),
            pltpu.VMEM((TPT,), jnp.float32),
            pltpu.VMEM((CCH, H), jnp.float32),
            pltpu.VMEM((CCH, H), jnp.float32),
            pltpu.VMEM((CCH, H), jnp.float32),
            pltpu.VMEM((L,), jnp.int32),
            pltpu.SemaphoreType.DMA,
        ],
    )(y_sorted, pos0, pos1, g0, g1, zeros16)


def kernel(x, router_weight, w1, w2):
    rw_pad = jnp.pad(router_weight, ((0, 0), (0, EPAD - E)))
    zeros16 = jnp.zeros((L,), jnp.int32)
    logits, start, blk = _router(x, rw_pad)
    be = blk[0, :NB]
    bu = blk[1, :NB]
    x_sorted, pos0, pos1, g0, g1 = _route_scatter(
        logits.reshape(N * EPAD), start, x, zeros16)
    y_sorted = _grouped_ffn(be, bu, x_sorted, w1, w2)
    return _combine(y_sorted, pos0, pos1, g0, g1, zeros16)
